# Initial kernel scaffold; baseline (speedup 1.0000x reference)
import jax
import jax.numpy as jnp
from jax.experimental import pallas as pl


def kernel(x, edge_index, W1, b1, W2, b2):
    raise NotImplementedError("write your pallas kernel here")


# speedup vs baseline: 40.7018x; 40.7018x over previous
import functools

import jax
import jax.numpy as jnp
from jax import lax
from jax.experimental import pallas as pl
from jax.experimental.pallas import tpu as pltpu
from jax.experimental.pallas import tpu_sc as plsc

N = 100000
E = 6400000
NP = 100352
RPW = NP // 16
EROWS = E // 128
JROWS = 8
CHUNK = JROWS * 128
ITERS = EROWS // (32 * JROWS)
TAIL0 = 32 * JROWS * ITERS
TAILW = (EROWS - TAIL0) // JROWS
BLK = 1024


def _seg_sum_sc(table, src2d, dst2d, zeros, width):
    mesh = plsc.VectorSubcoreMesh(core_axis_name="c", subcore_axis_name="s",
                                  num_cores=2, num_subcores=16)

    @functools.partial(
        pl.kernel,
        out_type=jax.ShapeDtypeStruct((2, NP, width), jnp.float32),
        mesh=mesh,
        scratch_types=dict(
            sidx=pltpu.VMEM((JROWS, 128), jnp.int32),
            didx=pltpu.VMEM((JROWS, 128), jnp.int32),
            rows=pltpu.VMEM((CHUNK, width), jnp.float32),
            acc=pltpu.VMEM_SHARED((NP, width), jnp.float32),
            sem=pltpu.SemaphoreType.DMA,
        ),
        compiler_params=pltpu.CompilerParams(use_tc_tiling_on_sc=False),
    )
    def body(table_hbm, src_hbm, dst_hbm, zero_hbm, out_hbm,
             sidx, didx, rows, acc, sem):
        c = lax.axis_index("c")
        s = lax.axis_index("s")
        wid = s * 2 + c

        pltpu.sync_copy(zero_hbm.at[pl.ds(s * RPW, RPW)],
                        acc.at[pl.ds(s * RPW, RPW)])
        plsc.subcore_barrier()

        def do_chunk(r0):
            pltpu.sync_copy(src_hbm.at[pl.ds(r0, JROWS)], sidx)
            pltpu.sync_copy(dst_hbm.at[pl.ds(r0, JROWS)], didx)
            cps = [
                pltpu.async_copy(table_hbm.at[sidx.at[j]],
                                 rows.at[pl.ds(j * 128, 128)], sem)
                for j in range(JROWS)
            ]
            for cp in cps:
                cp.wait()
            for j in range(JROWS):
                pltpu.sync_copy(rows.at[pl.ds(j * 128, 128)],
                                acc.at[didx.at[j]], add=True)

        @pl.loop(0, ITERS)
        def _edge_loop(i):
            do_chunk(wid * (JROWS * ITERS) + i * JROWS)

        @pl.when(wid < TAILW)
        def _tail():
            do_chunk(TAIL0 + wid * JROWS)

        plsc.subcore_barrier()
        pltpu.sync_copy(acc.at[pl.ds(s * RPW, RPW)],
                        out_hbm.at[c, pl.ds(s * RPW, RPW)])

    return body(table, src2d, dst2d, zeros)


def _mid_tc(p, w1p, b1p, w2p):
    def body(p_ref, w1_ref, b1_ref, w2_ref, o_ref):
        agg = p_ref[0] + p_ref[1]
        h = jnp.maximum(
            jnp.dot(agg, w1_ref[...], preferred_element_type=jnp.float32)
            + b1_ref[...], 0.0)
        o_ref[...] = jnp.dot(h, w2_ref[...],
                             preferred_element_type=jnp.float32)

    return pl.pallas_call(
        body,
        grid=(NP // BLK,),
        in_specs=[
            pl.BlockSpec((2, BLK, 8), lambda i: (0, i, 0)),
            pl.BlockSpec((8, 32), lambda i: (0, 0)),
            pl.BlockSpec((1, 32), lambda i: (0, 0)),
            pl.BlockSpec((32, 8), lambda i: (0, 0)),
        ],
        out_specs=pl.BlockSpec((BLK, 8), lambda i: (i, 0)),
        out_shape=jax.ShapeDtypeStruct((NP, 8), jnp.float32),
    )(p, w1p, b1p, w2p)


def _fin_tc(q, b2p):
    def body(q_ref, b2_ref, o_ref):
        o_ref[...] = q_ref[0] + q_ref[1] + b2_ref[...]

    return pl.pallas_call(
        body,
        grid=(NP // BLK,),
        in_specs=[
            pl.BlockSpec((2, BLK, 8), lambda i: (0, i, 0)),
            pl.BlockSpec((1, 8), lambda i: (0, 0)),
        ],
        out_specs=pl.BlockSpec((BLK, 8), lambda i: (i, 0)),
        out_shape=jax.ShapeDtypeStruct((NP, 8), jnp.float32),
    )(q, b2p)


def kernel(x, edge_index, W1, b1, W2, b2):
    xp = jnp.zeros((NP, 8), jnp.float32).at[:N, :6].set(x)
    src2d = edge_index[0].reshape(EROWS, 128)
    dst2d = edge_index[1].reshape(EROWS, 128)
    zeros8 = jnp.zeros((NP, 8), jnp.float32)
    w1p = jnp.zeros((8, 32), jnp.float32).at[:6, :].set(W1)
    b1p = b1.reshape(1, 32)
    w2p = jnp.zeros((32, 8), jnp.float32).at[:, :3].set(W2)
    b2p = jnp.zeros((1, 8), jnp.float32).at[0, :3].set(b2)

    p = _seg_sum_sc(xp, src2d, dst2d, zeros8, 8)
    h2 = _mid_tc(p, w1p, b1p, w2p)
    q = _seg_sum_sc(h2, src2d, dst2d, zeros8, 8)
    outp = _fin_tc(q, b2p)
    return outp[:N, :3]


# speedup vs baseline: 44.8121x; 1.1010x over previous
import functools

import jax
import jax.numpy as jnp
from jax import lax
from jax.experimental import pallas as pl
from jax.experimental.pallas import tpu as pltpu
from jax.experimental.pallas import tpu_sc as plsc

N = 100000
E = 6400000
NP = 100352
RPW = NP // 16
EROWS = E // 128
JROWS = 8
CHUNK = JROWS * 128
ITERS = EROWS // (32 * JROWS)
TAIL0 = 32 * JROWS * ITERS
TAILW = (EROWS - TAIL0) // JROWS
BLK = 1024


def _seg_sum_sc(table, src2d, dst2d, zeros, width):
    mesh = plsc.VectorSubcoreMesh(core_axis_name="c", subcore_axis_name="s",
                                  num_cores=2, num_subcores=16)

    @functools.partial(
        pl.kernel,
        out_type=jax.ShapeDtypeStruct((2, NP, width), jnp.float32),
        mesh=mesh,
        scratch_types=dict(
            sidx=pltpu.VMEM((JROWS, 128), jnp.int32),
            didx=pltpu.VMEM((JROWS, 128), jnp.int32),
            rows=pltpu.VMEM((CHUNK, width), jnp.float32),
            acc=pltpu.VMEM_SHARED((NP, width), jnp.float32),
            sem_a=pltpu.SemaphoreType.DMA,
            sem_b=pltpu.SemaphoreType.DMA,
        ),
        compiler_params=pltpu.CompilerParams(use_tc_tiling_on_sc=False),
    )
    def body(table_hbm, src_hbm, dst_hbm, zero_hbm, out_hbm,
             sidx, didx, rows, acc, sem_a, sem_b):
        c = lax.axis_index("c")
        s = lax.axis_index("s")
        wid = s * 2 + c

        pltpu.sync_copy(zero_hbm.at[pl.ds(s * RPW, RPW)],
                        acc.at[pl.ds(s * RPW, RPW)])
        plsc.subcore_barrier()

        half = JROWS // 2

        def do_chunk(r0):
            pltpu.sync_copy(src_hbm.at[pl.ds(r0, JROWS)], sidx)
            pltpu.sync_copy(dst_hbm.at[pl.ds(r0, JROWS)], didx)
            cps_a = [
                pltpu.async_copy(table_hbm.at[sidx.at[j]],
                                 rows.at[pl.ds(j * 128, 128)], sem_a)
                for j in range(half)
            ]
            cps_b = [
                pltpu.async_copy(table_hbm.at[sidx.at[j]],
                                 rows.at[pl.ds(j * 128, 128)], sem_b)
                for j in range(half, JROWS)
            ]
            for cp in cps_a:
                cp.wait()
            for j in range(half):
                pltpu.sync_copy(rows.at[pl.ds(j * 128, 128)],
                                acc.at[didx.at[j]], add=True)
            for cp in cps_b:
                cp.wait()
            for j in range(half, JROWS):
                pltpu.sync_copy(rows.at[pl.ds(j * 128, 128)],
                                acc.at[didx.at[j]], add=True)

        @pl.loop(0, ITERS)
        def _edge_loop(i):
            do_chunk(wid * (JROWS * ITERS) + i * JROWS)

        @pl.when(wid < TAILW)
        def _tail():
            do_chunk(TAIL0 + wid * JROWS)

        plsc.subcore_barrier()
        pltpu.sync_copy(acc.at[pl.ds(s * RPW, RPW)],
                        out_hbm.at[c, pl.ds(s * RPW, RPW)])

    return body(table, src2d, dst2d, zeros)


def _mid_tc(p, w1p, b1p, w2p):
    def body(p_ref, w1_ref, b1_ref, w2_ref, o_ref):
        agg = p_ref[0] + p_ref[1]
        h = jnp.maximum(
            jnp.dot(agg, w1_ref[...], preferred_element_type=jnp.float32)
            + b1_ref[...], 0.0)
        o_ref[...] = jnp.dot(h, w2_ref[...],
                             preferred_element_type=jnp.float32)

    return pl.pallas_call(
        body,
        grid=(NP // BLK,),
        in_specs=[
            pl.BlockSpec((2, BLK, 8), lambda i: (0, i, 0)),
            pl.BlockSpec((8, 32), lambda i: (0, 0)),
            pl.BlockSpec((1, 32), lambda i: (0, 0)),
            pl.BlockSpec((32, 8), lambda i: (0, 0)),
        ],
        out_specs=pl.BlockSpec((BLK, 8), lambda i: (i, 0)),
        out_shape=jax.ShapeDtypeStruct((NP, 8), jnp.float32),
    )(p, w1p, b1p, w2p)


def _fin_tc(q, b2p):
    def body(q_ref, b2_ref, o_ref):
        o_ref[...] = q_ref[0] + q_ref[1] + b2_ref[...]

    return pl.pallas_call(
        body,
        grid=(NP // BLK,),
        in_specs=[
            pl.BlockSpec((2, BLK, 8), lambda i: (0, i, 0)),
            pl.BlockSpec((1, 8), lambda i: (0, 0)),
        ],
        out_specs=pl.BlockSpec((BLK, 8), lambda i: (i, 0)),
        out_shape=jax.ShapeDtypeStruct((NP, 8), jnp.float32),
    )(q, b2p)


def kernel(x, edge_index, W1, b1, W2, b2):
    xp = jnp.zeros((NP, 8), jnp.float32).at[:N, :6].set(x)
    src2d = edge_index[0].reshape(EROWS, 128)
    dst2d = edge_index[1].reshape(EROWS, 128)
    zeros8 = jnp.zeros((NP, 8), jnp.float32)
    w1p = jnp.zeros((8, 32), jnp.float32).at[:6, :].set(W1)
    b1p = b1.reshape(1, 32)
    w2p = jnp.zeros((32, 8), jnp.float32).at[:, :3].set(W2)
    b2p = jnp.zeros((1, 8), jnp.float32).at[0, :3].set(b2)

    p = _seg_sum_sc(xp, src2d, dst2d, zeros8, 8)
    h2 = _mid_tc(p, w1p, b1p, w2p)
    q = _seg_sum_sc(h2, src2d, dst2d, zeros8, 8)
    outp = _fin_tc(q, b2p)
    return outp[:N, :3]


# speedup vs baseline: 54.0212x; 1.2055x over previous
import functools

import jax
import jax.numpy as jnp
from jax import lax
from jax.experimental import pallas as pl
from jax.experimental.pallas import tpu as pltpu
from jax.experimental.pallas import tpu_sc as plsc

N = 100000
E = 6400000
NP = 100352
RPW = NP // 16
EROWS = E // 128
JROWS = 8
CHUNK = JROWS * 128
ITERS = EROWS // (32 * JROWS)
TAIL0 = 32 * JROWS * ITERS
TAILW = (EROWS - TAIL0) // JROWS
NV = NP // 16
BLKV = 784


def _seg_sum_sc(table, ei3, zeros):
    mesh = plsc.VectorSubcoreMesh(core_axis_name="c", subcore_axis_name="s",
                                  num_cores=2, num_subcores=16)

    @functools.partial(
        pl.kernel,
        out_type=jax.ShapeDtypeStruct((2, NP, 8), jnp.float32),
        mesh=mesh,
        scratch_types=dict(
            sidx=pltpu.VMEM((JROWS, 128), jnp.int32),
            didx=pltpu.VMEM((JROWS, 128), jnp.int32),
            rows=pltpu.VMEM((CHUNK, 8), jnp.float32),
            acc=pltpu.VMEM_SHARED((NP, 8), jnp.float32),
            sem_a=pltpu.SemaphoreType.DMA,
            sem_b=pltpu.SemaphoreType.DMA,
        ),
        compiler_params=pltpu.CompilerParams(use_tc_tiling_on_sc=False),
    )
    def body(table_hbm, ei_hbm, zero_hbm, out_hbm,
             sidx, didx, rows, acc, sem_a, sem_b):
        c = lax.axis_index("c")
        s = lax.axis_index("s")
        wid = s * 2 + c

        pltpu.sync_copy(zero_hbm.at[pl.ds(s * RPW, RPW)],
                        acc.at[pl.ds(s * RPW, RPW)])
        plsc.subcore_barrier()

        half = JROWS // 2

        def do_chunk(r0):
            pltpu.sync_copy(ei_hbm.at[0, pl.ds(r0, JROWS)], sidx)
            pltpu.sync_copy(ei_hbm.at[1, pl.ds(r0, JROWS)], didx)
            cps_a = [
                pltpu.async_copy(table_hbm.at[sidx.at[j]],
                                 rows.at[pl.ds(j * 128, 128)], sem_a)
                for j in range(half)
            ]
            cps_b = [
                pltpu.async_copy(table_hbm.at[sidx.at[j]],
                                 rows.at[pl.ds(j * 128, 128)], sem_b)
                for j in range(half, JROWS)
            ]
            for cp in cps_a:
                cp.wait()
            for j in range(half):
                pltpu.sync_copy(rows.at[pl.ds(j * 128, 128)],
                                acc.at[didx.at[j]], add=True)
            for cp in cps_b:
                cp.wait()
            for j in range(half, JROWS):
                pltpu.sync_copy(rows.at[pl.ds(j * 128, 128)],
                                acc.at[didx.at[j]], add=True)

        @pl.loop(0, ITERS)
        def _edge_loop(i):
            do_chunk(wid * (JROWS * ITERS) + i * JROWS)

        @pl.when(wid < TAILW)
        def _tail():
            do_chunk(TAIL0 + wid * JROWS)

        plsc.subcore_barrier()
        pltpu.sync_copy(acc.at[pl.ds(s * RPW, RPW)],
                        out_hbm.at[c, pl.ds(s * RPW, RPW)])

    return body(table, ei3, zeros)


def _mid_tc(p, w1r, b1r, w2r):
    def body(p_ref, w1_ref, b1_ref, w2_ref, o_ref):
        agg = p_ref[0] + p_ref[1]
        h = jnp.maximum(
            jnp.dot(agg, w1_ref[...], preferred_element_type=jnp.float32)
            + b1_ref[...], 0.0)
        o_ref[...] = jnp.dot(h, w2_ref[...],
                             preferred_element_type=jnp.float32)

    return pl.pallas_call(
        body,
        grid=(NV // BLKV,),
        in_specs=[
            pl.BlockSpec((2, BLKV, 128), lambda i: (0, i, 0)),
            pl.BlockSpec((128, 512), lambda i: (0, 0)),
            pl.BlockSpec((1, 512), lambda i: (0, 0)),
            pl.BlockSpec((512, 128), lambda i: (0, 0)),
        ],
        out_specs=pl.BlockSpec((BLKV, 128), lambda i: (i, 0)),
        out_shape=jax.ShapeDtypeStruct((NV, 128), jnp.float32),
    )(p, w1r, b1r, w2r)


def _fin_tc(q, b2r):
    def body(q_ref, b2_ref, o_ref):
        o_ref[...] = q_ref[0] + q_ref[1] + b2_ref[...]

    return pl.pallas_call(
        body,
        grid=(NV // BLKV,),
        in_specs=[
            pl.BlockSpec((2, BLKV, 128), lambda i: (0, i, 0)),
            pl.BlockSpec((1, 128), lambda i: (0, 0)),
        ],
        out_specs=pl.BlockSpec((BLKV, 128), lambda i: (i, 0)),
        out_shape=jax.ShapeDtypeStruct((NV, 128), jnp.float32),
    )(q, b2r)


def kernel(x, edge_index, W1, b1, W2, b2):
    xp = jnp.zeros((NP, 8), jnp.float32).at[:N, :6].set(x)
    ei3 = edge_index.reshape(2, EROWS, 128)
    zeros8 = jnp.zeros((NP, 8), jnp.float32)
    eye16 = jnp.eye(16, dtype=jnp.float32)
    w1p = jnp.zeros((8, 32), jnp.float32).at[:6, :].set(W1)
    w2p = jnp.zeros((32, 8), jnp.float32).at[:, :3].set(W2)
    w1r = jnp.kron(eye16, w1p)
    w2r = jnp.kron(eye16, w2p)
    b1r = jnp.tile(b1, 16).reshape(1, 512)
    b2p = jnp.zeros((8,), jnp.float32).at[:3].set(b2)
    b2r = jnp.tile(b2p, 16).reshape(1, 128)

    p = _seg_sum_sc(xp, ei3, zeros8)
    h2 = _mid_tc(p.reshape(2, NV, 128), w1r, b1r, w2r)
    q = _seg_sum_sc(h2.reshape(NP, 8), ei3, zeros8)
    outp = _fin_tc(q.reshape(2, NV, 128), b2r)
    return outp.reshape(NP, 8)[:N, :3]


# speedup vs baseline: 65.2809x; 1.2084x over previous
import functools

import jax
import jax.numpy as jnp
from jax import lax
from jax.experimental import pallas as pl
from jax.experimental.pallas import tpu as pltpu
from jax.experimental.pallas import tpu_sc as plsc

N = 100000
E = 6400000
NP = 100352
RPW = NP // 16
EROWS = E // 128
JROWS = 16
CHUNK = JROWS * 128
ITERS = EROWS // (32 * JROWS)
TAIL0 = 32 * JROWS * ITERS
TAILW = (EROWS - TAIL0) // JROWS
NV = NP // 16
BLKV = 784


def _seg_sum_sc(table, ei3, zeros):
    mesh = plsc.VectorSubcoreMesh(core_axis_name="c", subcore_axis_name="s",
                                  num_cores=2, num_subcores=16)

    @functools.partial(
        pl.kernel,
        out_type=jax.ShapeDtypeStruct((2, NP, 8), jnp.float32),
        mesh=mesh,
        scratch_types=dict(
            sidx=pltpu.VMEM((JROWS, 128), jnp.int32),
            didx=pltpu.VMEM((JROWS, 128), jnp.int32),
            rows=pltpu.VMEM((CHUNK, 8), jnp.float32),
            acc=pltpu.VMEM_SHARED((NP, 8), jnp.float32),
            sem_a=pltpu.SemaphoreType.DMA,
            sem_b=pltpu.SemaphoreType.DMA,
        ),
        compiler_params=pltpu.CompilerParams(use_tc_tiling_on_sc=False),
    )
    def body(table_hbm, ei_hbm, zero_hbm, out_hbm,
             sidx, didx, rows, acc, sem_a, sem_b):
        c = lax.axis_index("c")
        s = lax.axis_index("s")
        wid = s * 2 + c

        pltpu.sync_copy(zero_hbm.at[pl.ds(s * RPW, RPW)],
                        acc.at[pl.ds(s * RPW, RPW)])
        plsc.subcore_barrier()

        half = JROWS // 2

        def do_chunk(r0):
            pltpu.sync_copy(ei_hbm.at[0, pl.ds(r0, JROWS)], sidx)
            pltpu.sync_copy(ei_hbm.at[1, pl.ds(r0, JROWS)], didx)
            cps_a = [
                pltpu.async_copy(table_hbm.at[sidx.at[j]],
                                 rows.at[pl.ds(j * 128, 128)], sem_a)
                for j in range(half)
            ]
            cps_b = [
                pltpu.async_copy(table_hbm.at[sidx.at[j]],
                                 rows.at[pl.ds(j * 128, 128)], sem_b)
                for j in range(half, JROWS)
            ]
            for cp in cps_a:
                cp.wait()
            for j in range(half):
                pltpu.sync_copy(rows.at[pl.ds(j * 128, 128)],
                                acc.at[didx.at[j]], add=True)
            for cp in cps_b:
                cp.wait()
            for j in range(half, JROWS):
                pltpu.sync_copy(rows.at[pl.ds(j * 128, 128)],
                                acc.at[didx.at[j]], add=True)

        @pl.loop(0, ITERS)
        def _edge_loop(i):
            do_chunk(wid * (JROWS * ITERS) + i * JROWS)

        @pl.when(wid < TAILW)
        def _tail():
            do_chunk(TAIL0 + wid * JROWS)

        plsc.subcore_barrier()
        pltpu.sync_copy(acc.at[pl.ds(s * RPW, RPW)],
                        out_hbm.at[c, pl.ds(s * RPW, RPW)])

    return body(table, ei3, zeros)


def _mid_tc(p, w1r, b1r, w2r):
    def body(p_ref, w1_ref, b1_ref, w2_ref, o_ref):
        agg = p_ref[0] + p_ref[1]
        h = jnp.maximum(
            jnp.dot(agg, w1_ref[...], preferred_element_type=jnp.float32)
            + b1_ref[...], 0.0)
        o_ref[...] = jnp.dot(h, w2_ref[...],
                             preferred_element_type=jnp.float32)

    return pl.pallas_call(
        body,
        grid=(NV // BLKV,),
        in_specs=[
            pl.BlockSpec((2, BLKV, 128), lambda i: (0, i, 0)),
            pl.BlockSpec((128, 512), lambda i: (0, 0)),
            pl.BlockSpec((1, 512), lambda i: (0, 0)),
            pl.BlockSpec((512, 128), lambda i: (0, 0)),
        ],
        out_specs=pl.BlockSpec((BLKV, 128), lambda i: (i, 0)),
        out_shape=jax.ShapeDtypeStruct((NV, 128), jnp.float32),
    )(p, w1r, b1r, w2r)


def _fin_tc(q, b2r):
    def body(q_ref, b2_ref, o_ref):
        o_ref[...] = q_ref[0] + q_ref[1] + b2_ref[...]

    return pl.pallas_call(
        body,
        grid=(NV // BLKV,),
        in_specs=[
            pl.BlockSpec((2, BLKV, 128), lambda i: (0, i, 0)),
            pl.BlockSpec((1, 128), lambda i: (0, 0)),
        ],
        out_specs=pl.BlockSpec((BLKV, 128), lambda i: (i, 0)),
        out_shape=jax.ShapeDtypeStruct((NV, 128), jnp.float32),
    )(q, b2r)


def kernel(x, edge_index, W1, b1, W2, b2):
    xp = jnp.zeros((NP, 8), jnp.float32).at[:N, :6].set(x)
    ei3 = edge_index.reshape(2, EROWS, 128)
    zeros8 = jnp.zeros((NP, 8), jnp.float32)
    eye16 = jnp.eye(16, dtype=jnp.float32)
    w1p = jnp.zeros((8, 32), jnp.float32).at[:6, :].set(W1)
    w2p = jnp.zeros((32, 8), jnp.float32).at[:, :3].set(W2)
    w1r = jnp.kron(eye16, w1p)
    w2r = jnp.kron(eye16, w2p)
    b1r = jnp.tile(b1, 16).reshape(1, 512)
    b2p = jnp.zeros((8,), jnp.float32).at[:3].set(b2)
    b2r = jnp.tile(b2p, 16).reshape(1, 128)

    p = _seg_sum_sc(xp, ei3, zeros8)
    h2 = _mid_tc(p.reshape(2, NV, 128), w1r, b1r, w2r)
    q = _seg_sum_sc(h2.reshape(NP, 8), ei3, zeros8)
    outp = _fin_tc(q.reshape(2, NV, 128), b2r)
    return outp.reshape(NP, 8)[:N, :3]


# speedup vs baseline: 73.3538x; 1.1237x over previous
import functools

import jax
import jax.numpy as jnp
from jax import lax
from jax.experimental import pallas as pl
from jax.experimental.pallas import tpu as pltpu
from jax.experimental.pallas import tpu_sc as plsc

N = 100000
E = 6400000
NP = 100352
RPW = NP // 16
EROWS = E // 128
JROWS = 32
CHUNK = JROWS * 128
ITERS = EROWS // (32 * JROWS)
TAIL0 = 32 * JROWS * ITERS
TAILW = (EROWS - TAIL0) // JROWS
TAIL1 = TAIL0 + TAILW * JROWS
TREM = EROWS - TAIL1
NV = NP // 16
BLKV = 784


def _seg_sum_sc(table, ei3, zeros):
    mesh = plsc.VectorSubcoreMesh(core_axis_name="c", subcore_axis_name="s",
                                  num_cores=2, num_subcores=16)

    @functools.partial(
        pl.kernel,
        out_type=jax.ShapeDtypeStruct((2, NP, 8), jnp.float32),
        mesh=mesh,
        scratch_types=dict(
            sidx=pltpu.VMEM((JROWS, 128), jnp.int32),
            didx=pltpu.VMEM((JROWS, 128), jnp.int32),
            rows=pltpu.VMEM((CHUNK, 8), jnp.float32),
            acc=pltpu.VMEM_SHARED((NP, 8), jnp.float32),
            sem_a=pltpu.SemaphoreType.DMA,
            sem_b=pltpu.SemaphoreType.DMA,
        ),
        compiler_params=pltpu.CompilerParams(use_tc_tiling_on_sc=False),
    )
    def body(table_hbm, ei_hbm, zero_hbm, out_hbm,
             sidx, didx, rows, acc, sem_a, sem_b):
        c = lax.axis_index("c")
        s = lax.axis_index("s")
        wid = s * 2 + c

        pltpu.sync_copy(zero_hbm.at[pl.ds(s * RPW, RPW)],
                        acc.at[pl.ds(s * RPW, RPW)])
        plsc.subcore_barrier()

        def do_chunk(r0, nrows):
            half = nrows // 2
            pltpu.sync_copy(ei_hbm.at[0, pl.ds(r0, nrows)],
                            sidx.at[pl.ds(0, nrows)])
            pltpu.sync_copy(ei_hbm.at[1, pl.ds(r0, nrows)],
                            didx.at[pl.ds(0, nrows)])
            cps_a = [
                pltpu.async_copy(table_hbm.at[sidx.at[j]],
                                 rows.at[pl.ds(j * 128, 128)], sem_a)
                for j in range(half)
            ]
            cps_b = [
                pltpu.async_copy(table_hbm.at[sidx.at[j]],
                                 rows.at[pl.ds(j * 128, 128)], sem_b)
                for j in range(half, nrows)
            ]
            for cp in cps_a:
                cp.wait()
            for j in range(half):
                pltpu.sync_copy(rows.at[pl.ds(j * 128, 128)],
                                acc.at[didx.at[j]], add=True)
            for cp in cps_b:
                cp.wait()
            for j in range(half, nrows):
                pltpu.sync_copy(rows.at[pl.ds(j * 128, 128)],
                                acc.at[didx.at[j]], add=True)

        @pl.loop(0, ITERS)
        def _edge_loop(i):
            do_chunk(wid * (JROWS * ITERS) + i * JROWS, JROWS)

        @pl.when(wid < TAILW)
        def _tail():
            do_chunk(TAIL0 + wid * JROWS, JROWS)

        if TREM:
            @pl.when(wid == TAILW)
            def _tail2():
                do_chunk(TAIL1, TREM)

        plsc.subcore_barrier()
        pltpu.sync_copy(acc.at[pl.ds(s * RPW, RPW)],
                        out_hbm.at[c, pl.ds(s * RPW, RPW)])

    return body(table, ei3, zeros)


def _mid_tc(p, w1r, b1r, w2r):
    def body(p_ref, w1_ref, b1_ref, w2_ref, o_ref):
        agg = p_ref[0] + p_ref[1]
        h = jnp.maximum(
            jnp.dot(agg, w1_ref[...], preferred_element_type=jnp.float32)
            + b1_ref[...], 0.0)
        o_ref[...] = jnp.dot(h, w2_ref[...],
                             preferred_element_type=jnp.float32)

    return pl.pallas_call(
        body,
        grid=(NV // BLKV,),
        in_specs=[
            pl.BlockSpec((2, BLKV, 128), lambda i: (0, i, 0)),
            pl.BlockSpec((128, 512), lambda i: (0, 0)),
            pl.BlockSpec((1, 512), lambda i: (0, 0)),
            pl.BlockSpec((512, 128), lambda i: (0, 0)),
        ],
        out_specs=pl.BlockSpec((BLKV, 128), lambda i: (i, 0)),
        out_shape=jax.ShapeDtypeStruct((NV, 128), jnp.float32),
    )(p, w1r, b1r, w2r)


def _fin_tc(q, b2r):
    def body(q_ref, b2_ref, o_ref):
        o_ref[...] = q_ref[0] + q_ref[1] + b2_ref[...]

    return pl.pallas_call(
        body,
        grid=(NV // BLKV,),
        in_specs=[
            pl.BlockSpec((2, BLKV, 128), lambda i: (0, i, 0)),
            pl.BlockSpec((1, 128), lambda i: (0, 0)),
        ],
        out_specs=pl.BlockSpec((BLKV, 128), lambda i: (i, 0)),
        out_shape=jax.ShapeDtypeStruct((NV, 128), jnp.float32),
    )(q, b2r)


def kernel(x, edge_index, W1, b1, W2, b2):
    xp = jnp.zeros((NP, 8), jnp.float32).at[:N, :6].set(x)
    ei3 = edge_index.reshape(2, EROWS, 128)
    zeros8 = jnp.zeros((NP, 8), jnp.float32)
    eye16 = jnp.eye(16, dtype=jnp.float32)
    w1p = jnp.zeros((8, 32), jnp.float32).at[:6, :].set(W1)
    w2p = jnp.zeros((32, 8), jnp.float32).at[:, :3].set(W2)
    w1r = jnp.kron(eye16, w1p)
    w2r = jnp.kron(eye16, w2p)
    b1r = jnp.tile(b1, 16).reshape(1, 512)
    b2p = jnp.zeros((8,), jnp.float32).at[:3].set(b2)
    b2r = jnp.tile(b2p, 16).reshape(1, 128)

    p = _seg_sum_sc(xp, ei3, zeros8)
    h2 = _mid_tc(p.reshape(2, NV, 128), w1r, b1r, w2r)
    q = _seg_sum_sc(h2.reshape(NP, 8), ei3, zeros8)
    outp = _fin_tc(q.reshape(2, NV, 128), b2r)
    return outp.reshape(NP, 8)[:N, :3]


# speedup vs baseline: 79.5524x; 1.0845x over previous
import functools

import jax
import jax.numpy as jnp
from jax import lax
from jax.experimental import pallas as pl
from jax.experimental.pallas import tpu as pltpu
from jax.experimental.pallas import tpu_sc as plsc

N = 100000
E = 6400000
NP = 100352
RPW = NP // 16
EROWS = E // 128
JROWS = 48
CHUNK = JROWS * 128
ITERS = EROWS // (32 * JROWS)
TAIL0 = 32 * JROWS * ITERS
TAILW = (EROWS - TAIL0) // JROWS
TAIL1 = TAIL0 + TAILW * JROWS
TREM = EROWS - TAIL1
NV = NP // 16
BLKV = 784


def _seg_sum_sc(table, ei3, zeros):
    mesh = plsc.VectorSubcoreMesh(core_axis_name="c", subcore_axis_name="s",
                                  num_cores=2, num_subcores=16)

    @functools.partial(
        pl.kernel,
        out_type=jax.ShapeDtypeStruct((2, NP, 8), jnp.float32),
        mesh=mesh,
        scratch_types=dict(
            sidx=pltpu.VMEM((JROWS, 128), jnp.int32),
            didx=pltpu.VMEM((JROWS, 128), jnp.int32),
            rows=pltpu.VMEM((CHUNK, 8), jnp.float32),
            acc=pltpu.VMEM_SHARED((NP, 8), jnp.float32),
            sem_a=pltpu.SemaphoreType.DMA,
            sem_b=pltpu.SemaphoreType.DMA,
        ),
        compiler_params=pltpu.CompilerParams(use_tc_tiling_on_sc=False),
    )
    def body(table_hbm, ei_hbm, zero_hbm, out_hbm,
             sidx, didx, rows, acc, sem_a, sem_b):
        c = lax.axis_index("c")
        s = lax.axis_index("s")
        wid = s * 2 + c

        pltpu.sync_copy(zero_hbm.at[pl.ds(s * RPW, RPW)],
                        acc.at[pl.ds(s * RPW, RPW)])
        plsc.subcore_barrier()

        def do_chunk(r0, nrows):
            half = nrows // 2
            pltpu.sync_copy(ei_hbm.at[0, pl.ds(r0, nrows)],
                            sidx.at[pl.ds(0, nrows)])
            pltpu.sync_copy(ei_hbm.at[1, pl.ds(r0, nrows)],
                            didx.at[pl.ds(0, nrows)])
            cps_a = [
                pltpu.async_copy(table_hbm.at[sidx.at[j]],
                                 rows.at[pl.ds(j * 128, 128)], sem_a)
                for j in range(half)
            ]
            cps_b = [
                pltpu.async_copy(table_hbm.at[sidx.at[j]],
                                 rows.at[pl.ds(j * 128, 128)], sem_b)
                for j in range(half, nrows)
            ]
            for cp in cps_a:
                cp.wait()
            for j in range(half):
                pltpu.sync_copy(rows.at[pl.ds(j * 128, 128)],
                                acc.at[didx.at[j]], add=True)
            for cp in cps_b:
                cp.wait()
            for j in range(half, nrows):
                pltpu.sync_copy(rows.at[pl.ds(j * 128, 128)],
                                acc.at[didx.at[j]], add=True)

        @pl.loop(0, ITERS)
        def _edge_loop(i):
            do_chunk(wid * (JROWS * ITERS) + i * JROWS, JROWS)

        @pl.when(wid < TAILW)
        def _tail():
            do_chunk(TAIL0 + wid * JROWS, JROWS)

        if TREM:
            @pl.when(wid == TAILW)
            def _tail2():
                do_chunk(TAIL1, TREM)

        plsc.subcore_barrier()
        pltpu.sync_copy(acc.at[pl.ds(s * RPW, RPW)],
                        out_hbm.at[c, pl.ds(s * RPW, RPW)])

    return body(table, ei3, zeros)


def _mid_tc(p, w1r, b1r, w2r):
    def body(p_ref, w1_ref, b1_ref, w2_ref, o_ref):
        agg = p_ref[0] + p_ref[1]
        h = jnp.maximum(
            jnp.dot(agg, w1_ref[...], preferred_element_type=jnp.float32)
            + b1_ref[...], 0.0)
        o_ref[...] = jnp.dot(h, w2_ref[...],
                             preferred_element_type=jnp.float32)

    return pl.pallas_call(
        body,
        grid=(NV // BLKV,),
        in_specs=[
            pl.BlockSpec((2, BLKV, 128), lambda i: (0, i, 0)),
            pl.BlockSpec((128, 512), lambda i: (0, 0)),
            pl.BlockSpec((1, 512), lambda i: (0, 0)),
            pl.BlockSpec((512, 128), lambda i: (0, 0)),
        ],
        out_specs=pl.BlockSpec((BLKV, 128), lambda i: (i, 0)),
        out_shape=jax.ShapeDtypeStruct((NV, 128), jnp.float32),
    )(p, w1r, b1r, w2r)


def _fin_tc(q, b2r):
    def body(q_ref, b2_ref, o_ref):
        o_ref[...] = q_ref[0] + q_ref[1] + b2_ref[...]

    return pl.pallas_call(
        body,
        grid=(NV // BLKV,),
        in_specs=[
            pl.BlockSpec((2, BLKV, 128), lambda i: (0, i, 0)),
            pl.BlockSpec((1, 128), lambda i: (0, 0)),
        ],
        out_specs=pl.BlockSpec((BLKV, 128), lambda i: (i, 0)),
        out_shape=jax.ShapeDtypeStruct((NV, 128), jnp.float32),
    )(q, b2r)


def kernel(x, edge_index, W1, b1, W2, b2):
    xp = jnp.zeros((NP, 8), jnp.float32).at[:N, :6].set(x)
    ei3 = edge_index.reshape(2, EROWS, 128)
    zeros8 = jnp.zeros((NP, 8), jnp.float32)
    eye16 = jnp.eye(16, dtype=jnp.float32)
    w1p = jnp.zeros((8, 32), jnp.float32).at[:6, :].set(W1)
    w2p = jnp.zeros((32, 8), jnp.float32).at[:, :3].set(W2)
    w1r = jnp.kron(eye16, w1p)
    w2r = jnp.kron(eye16, w2p)
    b1r = jnp.tile(b1, 16).reshape(1, 512)
    b2p = jnp.zeros((8,), jnp.float32).at[:3].set(b2)
    b2r = jnp.tile(b2p, 16).reshape(1, 128)

    p = _seg_sum_sc(xp, ei3, zeros8)
    h2 = _mid_tc(p.reshape(2, NV, 128), w1r, b1r, w2r)
    q = _seg_sum_sc(h2.reshape(NP, 8), ei3, zeros8)
    outp = _fin_tc(q.reshape(2, NV, 128), b2r)
    return outp.reshape(NP, 8)[:N, :3]
